# baseline (speedup 1.0000x reference)
import functools

import jax
import jax.numpy as jnp
from jax import lax
from jax.experimental import pallas as pl
from jax.experimental.pallas import tpu as pltpu
from jax.experimental.pallas import tpu_sc as plsc

BATCH = 16384
N_FIELDS = 26
NC = 2
NS = 16
NW = NC * NS
B_PER_W = BATCH // NW
CHUNK = B_PER_W * N_FIELDS
LANES = 16
UNROLL = 8


@functools.partial(
    pl.kernel,
    out_type=jax.ShapeDtypeStruct((BATCH,), jnp.float32),
    mesh=plsc.VectorSubcoreMesh(core_axis_name="c", subcore_axis_name="s"),
    compiler_params=pltpu.CompilerParams(needs_layout_passes=False),
    scratch_types=[
        pltpu.VMEM((CHUNK,), jnp.int32),
        pltpu.VMEM((CHUNK,), jnp.float32),
        pltpu.VMEM((CHUNK,), jnp.float32),
        pltpu.VMEM((B_PER_W,), jnp.float32),
        pltpu.SemaphoreType.DMA,
    ],
)
def _sc_linear(idx_hbm, vals_hbm, w_hbm, out_hbm, idx_v, rows_v,
               vals_v, out_v, sem):
    w = lax.axis_index("c") * NS + lax.axis_index("s")
    pltpu.sync_copy(idx_hbm.at[w], idx_v)
    gather = pltpu.async_copy(w_hbm.at[idx_v], rows_v, sem)
    pltpu.sync_copy(vals_hbm.at[w], vals_v)
    gather.wait()

    lane = lax.iota(jnp.int32, LANES)
    tail_mask = lane >= (2 * LANES - N_FIELDS)

    def body(g, carry):
        acc = jnp.zeros((LANES,), jnp.float32)
        for u in range(LANES):
            base = (g * LANES + u) * N_FIELDS
            head = (rows_v[pl.ds(base, LANES)]
                    * vals_v[pl.ds(base, LANES)])
            tail = (rows_v[pl.ds(base + N_FIELDS - LANES, LANES)]
                    * vals_v[pl.ds(base + N_FIELDS - LANES, LANES)])
            s = jnp.sum(head + jnp.where(tail_mask, tail, 0.0))
            acc = jnp.where(lane == u, s, acc)
        out_v[pl.ds(g * LANES, LANES)] = acc
        return carry

    lax.fori_loop(0, B_PER_W // LANES, body, 0)
    pltpu.sync_copy(out_v, out_hbm.at[pl.ds(w * B_PER_W, B_PER_W)])


def kernel(feature_idx, feature_vals, W, b):
    idx = feature_idx.astype(jnp.int32).reshape(NW, CHUNK)
    vals = feature_vals.reshape(NW, CHUNK)
    return _sc_linear(idx, vals, W[:, 0]).reshape(BATCH, 1) + b


# speedup vs baseline: 1.0046x; 1.0046x over previous
import functools

import jax
import jax.numpy as jnp
from jax import lax
from jax.experimental import pallas as pl
from jax.experimental.pallas import tpu as pltpu
from jax.experimental.pallas import tpu_sc as plsc

BATCH = 16384
N_FIELDS = 26
NC = 2
NS = 16
NW = NC * NS
B_PER_W = BATCH // NW
CHUNK = B_PER_W * N_FIELDS
LANES = 16
N_VECS = B_PER_W // LANES


@functools.partial(
    pl.kernel,
    out_type=jax.ShapeDtypeStruct((BATCH,), jnp.float32),
    mesh=plsc.VectorSubcoreMesh(core_axis_name="c", subcore_axis_name="s"),
    compiler_params=pltpu.CompilerParams(needs_layout_passes=False),
    scratch_types=[
        pltpu.VMEM((CHUNK,), jnp.int32),
        pltpu.VMEM((CHUNK,), jnp.float32),
        pltpu.VMEM((CHUNK,), jnp.float32),
        pltpu.VMEM((B_PER_W,), jnp.float32),
        pltpu.SemaphoreType.DMA,
    ],
)
def _sc_linear(idx_hbm, vals_hbm, w_hbm, out_hbm, idx_v, rows_v, vals_v,
               out_v, sem):
    w = lax.axis_index("c") * NS + lax.axis_index("s")
    pltpu.sync_copy(idx_hbm.at[w], idx_v)
    gather = pltpu.async_copy(w_hbm.at[idx_v], rows_v, sem)
    pltpu.sync_copy(vals_hbm.at[w], vals_v)
    gather.wait()

    stride_iota = lax.iota(jnp.int32, LANES) * N_FIELDS

    def body(s, carry):
        flat_base = s * (LANES * N_FIELDS)
        acc = jnp.zeros((LANES,), jnp.float32)
        for j in range(N_FIELDS):
            idxv = stride_iota + (flat_base + j)
            acc = acc + (plsc.load_gather(rows_v, [idxv])
                         * plsc.load_gather(vals_v, [idxv]))
        out_v[pl.ds(s * LANES, LANES)] = acc
        return carry

    lax.fori_loop(0, N_VECS, body, 0)
    pltpu.sync_copy(out_v, out_hbm.at[pl.ds(w * B_PER_W, B_PER_W)])


def kernel(feature_idx, feature_vals, W, b):
    idx = feature_idx.astype(jnp.int32).reshape(NW, CHUNK)
    vals = feature_vals.reshape(NW, CHUNK)
    return _sc_linear(idx, vals, W[:, 0]).reshape(BATCH, 1) + b


# speedup vs baseline: 1.2017x; 1.1962x over previous
import functools

import jax
import jax.numpy as jnp
from jax import lax
from jax.experimental import pallas as pl
from jax.experimental.pallas import tpu as pltpu
from jax.experimental.pallas import tpu_sc as plsc

BATCH = 16384
N_FIELDS = 26
NC = 2
NS = 16
NW = NC * NS
B_PER_W = BATCH // NW
CHUNK = B_PER_W * N_FIELDS
LANES = 16
N_VECS = B_PER_W // LANES


@functools.partial(
    pl.kernel,
    out_type=jax.ShapeDtypeStruct((BATCH,), jnp.float32),
    mesh=plsc.VectorSubcoreMesh(core_axis_name="c", subcore_axis_name="s"),
    compiler_params=pltpu.CompilerParams(needs_layout_passes=False),
    scratch_types=[
        pltpu.VMEM((CHUNK,), jnp.int32),
        pltpu.VMEM((CHUNK,), jnp.float32),
        pltpu.VMEM((CHUNK,), jnp.float32),
        pltpu.VMEM((B_PER_W,), jnp.float32),
        pltpu.SemaphoreType.DMA,
    ],
)
def _sc_linear(idx_hbm, vals_hbm, w_hbm, out_hbm, idx_v, rows_v, vals_v,
               out_v, sem):
    w = lax.axis_index("c") * NS + lax.axis_index("s")
    pltpu.sync_copy(idx_hbm.at[w], idx_v)
    gather = pltpu.async_copy(w_hbm.at[idx_v], rows_v, sem)
    pltpu.sync_copy(vals_hbm.at[w], vals_v)
    gather.wait()

    def body(s, carry):
        base = s * LANES
        acc = jnp.zeros((LANES,), jnp.float32)
        for j in range(N_FIELDS):
            off = pl.ds(j * B_PER_W + base, LANES)
            acc = acc + rows_v[off] * vals_v[off]
        out_v[pl.ds(base, LANES)] = acc
        return carry

    lax.fori_loop(0, N_VECS, body, 0)
    pltpu.sync_copy(out_v, out_hbm.at[pl.ds(w * B_PER_W, B_PER_W)])


def kernel(feature_idx, feature_vals, W, b):
    idx = (feature_idx.astype(jnp.int32)
           .reshape(NW, B_PER_W, N_FIELDS).transpose(0, 2, 1)
           .reshape(NW, CHUNK))
    vals = (feature_vals.reshape(NW, B_PER_W, N_FIELDS).transpose(0, 2, 1)
            .reshape(NW, CHUNK))
    out = _sc_linear(idx, vals, W.reshape(-1))
    return out.reshape(BATCH, 1) + b
